# baseline (speedup 1.0000x reference)
import functools

import jax
import jax.numpy as jnp
from jax.experimental import pallas as pl
from jax.experimental.pallas import tpu as pltpu

_NUM_CLASSES = 81
_NEG_RATIO = 3
_CHUNK = 8000


def _stage_a(conf_ref, lab_ref, loc_ref, gt_ref, x_ref, np_ref, ll_ref, pc_ref):
    i = pl.program_id(0)
    conf = conf_ref[...]
    lab = lab_ref[...]

    m = jnp.max(conf, axis=1, keepdims=True)
    lse = jnp.log(jnp.sum(jnp.exp(conf - m), axis=1, keepdims=True)) + m
    lane = jax.lax.broadcasted_iota(jnp.int32, conf.shape, 1)
    sel = jnp.sum(jnp.where(lane == lab, conf, 0.0), axis=1, keepdims=True)
    closs = lse - sel

    pos = lab > 0
    posf = pos.astype(jnp.float32)

    d = loc_ref[...] - gt_ref[...]
    ad = jnp.abs(d)
    sl1 = jnp.where(ad < 1.0, 0.5 * d * d, ad - 0.5)
    loc_part = jnp.sum(jnp.where(pos, sl1, 0.0))

    np_part = jnp.sum(posf)
    pc_part = jnp.sum(closs * posf)

    x_ref[...] = jnp.where(pos, -1.0, closs)

    @pl.when(i == 0)
    def _init():
        np_ref[0, 0] = np_part
        ll_ref[0, 0] = loc_part
        pc_ref[0, 0] = pc_part

    @pl.when(i != 0)
    def _acc():
        np_ref[0, 0] += np_part
        ll_ref[0, 0] += loc_part
        pc_ref[0, 0] += pc_part


def _stage_b(x_ref, np_ref, ll_ref, pc_ref, out_ref):
    x = x_ref[...]
    num_pos = np_ref[0, 0]
    count_neg = jnp.sum((x >= 0.0).astype(jnp.float32))
    k = jnp.minimum(_NEG_RATIO * num_pos, count_neg)

    def body(_, carry):
        lo, hi = carry
        mid = lo + (hi - lo) // 2
        t = jax.lax.bitcast_convert_type(mid, jnp.float32)
        cnt = jnp.sum((x >= t).astype(jnp.float32))
        big = cnt >= k
        return jnp.where(big, mid, lo), jnp.where(big, hi, mid)

    lo, _ = jax.lax.fori_loop(
        0, 31, body, (jnp.int32(0), jnp.int32(0x7F800000)))
    t = jax.lax.bitcast_convert_type(lo, jnp.float32)

    gt_mask = x > t
    cnt_gt = jnp.sum(gt_mask.astype(jnp.float32))
    sum_gt = jnp.sum(jnp.where(gt_mask, x, 0.0))
    extra = k - cnt_gt
    neg_sum = sum_gt + jnp.where(extra > 0.0, extra * t, 0.0)

    out_ref[0, 0] = (ll_ref[0, 0] + pc_ref[0, 0] + neg_sum) / num_pos


@functools.partial(jax.jit, static_argnames=("interpret",))
def _run(pred_loc, pred_conf, gt_loc, gt_label, interpret=False):
    B, N, C = pred_conf.shape
    total = B * N
    conf2 = pred_conf.reshape(total, C)
    loc2 = pred_loc.reshape(total, 4)
    gt2 = gt_loc.reshape(total, 4)
    lab2 = gt_label.astype(jnp.int32).reshape(total, 1)

    steps = total // _CHUNK
    scal = jax.ShapeDtypeStruct((1, 1), jnp.float32)
    sspec = pl.BlockSpec((1, 1), lambda i: (0, 0), memory_space=pltpu.SMEM)

    x, np_, ll, pc = pl.pallas_call(
        _stage_a,
        grid=(steps,),
        in_specs=[
            pl.BlockSpec((_CHUNK, C), lambda i: (i, 0)),
            pl.BlockSpec((_CHUNK, 1), lambda i: (i, 0)),
            pl.BlockSpec((_CHUNK, 4), lambda i: (i, 0)),
            pl.BlockSpec((_CHUNK, 4), lambda i: (i, 0)),
        ],
        out_specs=[
            pl.BlockSpec((_CHUNK, 1), lambda i: (i, 0)),
            sspec, sspec, sspec,
        ],
        out_shape=[
            jax.ShapeDtypeStruct((total, 1), jnp.float32),
            scal, scal, scal,
        ],
        interpret=interpret,
    )(conf2, lab2, loc2, gt2)

    xr = x.reshape(total // 128, 128)
    out = pl.pallas_call(
        _stage_b,
        in_specs=[
            pl.BlockSpec((total // 128, 128), lambda: (0, 0)),
            pl.BlockSpec(memory_space=pltpu.SMEM),
            pl.BlockSpec(memory_space=pltpu.SMEM),
            pl.BlockSpec(memory_space=pltpu.SMEM),
        ],
        out_specs=pl.BlockSpec((1, 1), lambda: (0, 0), memory_space=pltpu.SMEM),
        out_shape=jax.ShapeDtypeStruct((1, 1), jnp.float32),
        interpret=interpret,
    )(xr, np_, ll, pc)
    return out.reshape(())


def kernel(pred_loc, pred_conf, gt_loc, gt_label):
    return _run(pred_loc, pred_conf, gt_loc, gt_label)
